# baseline (speedup 1.0000x reference)
import functools

import jax
import jax.numpy as jnp
from jax import lax
from jax.experimental import pallas as pl
from jax.experimental.pallas import tpu as pltpu
from jax.experimental.pallas import tpu_sc as plsc

N = 10000
D = 128
H = 64
E = 320000
NTILES = 16
L = 128
B = 32
NB = 5
NSLOT = 4
K = B * NB
EPAD = NTILES * K * L - E
ACC_ROWS = 10240


def _sc_body(src_go, dst_go, src_bk, dst_bk, fu0, fu1, fi0, fi1,
             agg_i0, agg_i1, agg_u0, agg_u1, sidx, didx, *scratch):
    rows = scratch[:NSLOT]
    tbl = scratch[NSLOT]
    acc = scratch[NSLOT + 1]
    gsems = scratch[NSLOT + 2:2 * NSLOT + 2]
    ssems = scratch[2 * NSLOT + 2:]
    c = lax.axis_index("c")
    s = lax.axis_index("s")
    rows0 = rows[0]

    zv = jnp.zeros((16,), jnp.float32)

    def zbody(i, carry):
        rows0[i // 4, pl.ds((i % 4) * 16, 16)] = zv
        return carry

    def wait_g(t):
        pltpu.make_async_copy(tbl.at[sidx.at[0]], rows[t], gsems[t]).wait()

    def wait_s(t):
        pltpu.make_async_copy(rows[t], acc.at[didx.at[0]], ssems[t]).wait()

    def half(src_e, dst_e, feat_h, out_hbm):
        base = s * 640
        pltpu.sync_copy(feat_h.at[pl.ds(base, 640)], tbl.at[pl.ds(base, 640)])
        lax.fori_loop(0, L * 4, zbody, 0)
        for k in range(640 // L):
            pltpu.sync_copy(rows0, acc.at[pl.ds(base + k * L, L)])
        plsc.subcore_barrier()

        for b in range(NB):
            pltpu.sync_copy(src_e.at[s, pl.ds(b * B, B)], sidx)
            pltpu.sync_copy(dst_e.at[s, pl.ds(b * B, B)], didx)
            for t in range(NSLOT):
                pltpu.async_copy(tbl.at[sidx.at[t]], rows[t], gsems[t])

            def grp(q, carry):
                j0 = q * NSLOT
                for t in range(NSLOT):
                    wait_g(t)
                    pltpu.async_copy(rows[t], acc.at[didx.at[j0 + t]], ssems[t], add=True)
                for t in range(NSLOT):
                    wait_s(t)
                    pltpu.async_copy(tbl.at[sidx.at[j0 + NSLOT + t]], rows[t], gsems[t])
                return carry

            lax.fori_loop(0, B // NSLOT - 1, grp, 0)
            for t in range(NSLOT):
                wait_g(t)
                pltpu.async_copy(rows[t], acc.at[didx.at[B - NSLOT + t]], ssems[t], add=True)
            for t in range(NSLOT):
                wait_s(t)
        plsc.subcore_barrier()

        for k in range(640 // L):
            pltpu.sync_copy(acc.at[pl.ds(base + k * L, L)], rows0)
            pltpu.sync_copy(rows0, out_hbm.at[pl.ds(base + k * L, L)])
        plsc.subcore_barrier()

    @pl.when(c == 0)
    def _():
        half(src_go, dst_go, fu0, agg_i0)
        half(src_go, dst_go, fu1, agg_i1)

    @pl.when(c == 1)
    def _():
        half(src_bk, dst_bk, fi0, agg_u0)
        half(src_bk, dst_bk, fi1, agg_u1)


_sc_agg = functools.partial(
    pl.kernel,
    out_type=[jax.ShapeDtypeStruct((ACC_ROWS, H), jnp.float32)] * 4,
    mesh=plsc.VectorSubcoreMesh(core_axis_name="c", subcore_axis_name="s"),
    compiler_params=pltpu.CompilerParams(use_tc_tiling_on_sc=False),
    scratch_types=[
        pltpu.VMEM((B, L), jnp.int32),
        pltpu.VMEM((B, L), jnp.int32),
        *[pltpu.VMEM((L, H), jnp.float32) for _ in range(NSLOT)],
        pltpu.VMEM_SHARED((ACC_ROWS, H), jnp.float32),
        pltpu.VMEM_SHARED((ACC_ROWS, H), jnp.float32),
        *[pltpu.SemaphoreType.DMA for _ in range(2 * NSLOT)],
    ],
)(_sc_body)


def _halves(x):
    xp = jnp.pad(x, ((0, ACC_ROWS - N), (0, 0)))
    return xp[:, :H], xp[:, H:]


def _sc_layer(src_go, dst_go, src_bk, dst_bk, user_feat, item_feat):
    fu0, fu1 = _halves(user_feat)
    fi0, fi1 = _halves(item_feat)
    a_i0, a_i1, a_u0, a_u1 = _sc_agg(src_go, dst_go, src_bk, dst_bk,
                                     fu0, fu1, fi0, fi1)
    agg_i = jnp.concatenate([a_i0[:N], a_i1[:N]], axis=1)
    agg_u = jnp.concatenate([a_u0[:N], a_u1[:N]], axis=1)
    return agg_i, agg_u


def _dense_body(eps_ref, agg_ref, feat_ref, W_ref, b_ref, g_ref, be_ref, out_ref):
    x = (1.0 + eps_ref[0, 0]) * feat_ref[:] + agg_ref[:]
    h = jnp.dot(x, W_ref[:], preferred_element_type=jnp.float32) + b_ref[:]
    m = jnp.mean(h, axis=0, keepdims=True)
    v = jnp.mean((h - m) ** 2, axis=0, keepdims=True)
    xn = (h - m) * lax.rsqrt(v + 1e-5) * g_ref[:] + be_ref[:]
    out_ref[:] = xn * jax.nn.sigmoid(xn)


_dense = pl.pallas_call(
    _dense_body,
    out_shape=jax.ShapeDtypeStruct((N, D), jnp.float32),
)


def _dense_layer(eps, agg, feat, W, b, gamma, beta):
    return _dense(
        jnp.reshape(eps, (1, 1)),
        agg,
        feat,
        W,
        jnp.reshape(b, (1, D)),
        jnp.reshape(gamma, (1, D)),
        jnp.reshape(beta, (1, D)),
    )


def kernel(edge_index_go, edge_index_back, user_emb, item_emb,
           W1_go, b1_go, W1_back, b1_back, W2_go, b2_go, W2_back, b2_back,
           eps1_go, eps1_back, eps2_go, eps2_back,
           gamma1_user, beta1_user, gamma1_item, beta1_item,
           gamma2_user, beta2_user, gamma2_item, beta2_item):
    def prep(col, fill):
        col = col.astype(jnp.int32)
        return jnp.pad(col, (0, EPAD), constant_values=fill).reshape(NTILES, K, L)

    src_go = prep(edge_index_go[0], 0)
    dst_go = prep(edge_index_go[1], N)
    src_bk = prep(edge_index_back[0], 0)
    dst_bk = prep(edge_index_back[1], N)

    agg_i1, agg_u1 = _sc_layer(src_go, dst_go, src_bk, dst_bk, user_emb, item_emb)
    item1 = _dense_layer(eps1_go, agg_i1, item_emb, W1_go, b1_go, gamma1_item, beta1_item)
    user1 = _dense_layer(eps1_back, agg_u1, user_emb, W1_back, b1_back, gamma1_user, beta1_user)

    agg_i2, agg_u2 = _sc_layer(src_go, dst_go, src_bk, dst_bk, user1, item1)
    item2 = _dense_layer(eps2_go, agg_i2, item1, W2_go, b2_go, gamma2_item, beta2_item)
    user2 = _dense_layer(eps2_back, agg_u2, user1, W2_back, b2_back, gamma2_user, beta2_user)

    return jnp.concatenate([user2, item2], axis=0)


# speedup vs baseline: 1.0801x; 1.0801x over previous
import functools

import jax
import jax.numpy as jnp
from jax import lax
from jax.experimental import pallas as pl
from jax.experimental.pallas import tpu as pltpu
from jax.experimental.pallas import tpu_sc as plsc

N = 10000
D = 128
H = 64
E = 320000
NTILES = 16
L = 64
B = 64
NB = 5
NSLOT = 8
K = B * NB
EPAD = NTILES * K * L - E
ACC_ROWS = 10240


def _sc_body(src_go, dst_go, src_bk, dst_bk, fu0, fu1, fi0, fi1,
             agg_i0, agg_i1, agg_u0, agg_u1, sidx, didx, *scratch):
    rows = scratch[:NSLOT]
    tbl = scratch[NSLOT]
    acc = scratch[NSLOT + 1]
    gsems = scratch[NSLOT + 2:2 * NSLOT + 2]
    ssems = scratch[2 * NSLOT + 2:]
    c = lax.axis_index("c")
    s = lax.axis_index("s")
    rows0 = rows[0]

    zv = jnp.zeros((16,), jnp.float32)

    def zbody(i, carry):
        rows0[i // 4, pl.ds((i % 4) * 16, 16)] = zv
        return carry

    def wait_g(t):
        pltpu.make_async_copy(tbl.at[sidx.at[0]], rows[t], gsems[t]).wait()

    def wait_s(t):
        pltpu.make_async_copy(rows[t], acc.at[didx.at[0]], ssems[t]).wait()

    def half(src_e, dst_e, feat_h, out_hbm):
        base = s * 640
        pltpu.sync_copy(feat_h.at[pl.ds(base, 640)], tbl.at[pl.ds(base, 640)])
        lax.fori_loop(0, L * 4, zbody, 0)
        for k in range(640 // L):
            pltpu.sync_copy(rows0, acc.at[pl.ds(base + k * L, L)])
        plsc.subcore_barrier()

        for b in range(NB):
            pltpu.sync_copy(src_e.at[s, pl.ds(b * B, B)], sidx)
            pltpu.sync_copy(dst_e.at[s, pl.ds(b * B, B)], didx)
            for t in range(NSLOT):
                pltpu.async_copy(tbl.at[sidx.at[t]], rows[t], gsems[t])

            def grp(q, carry):
                j0 = q * NSLOT
                for t in range(NSLOT):
                    wait_g(t)
                    pltpu.async_copy(rows[t], acc.at[didx.at[j0 + t]], ssems[t], add=True)
                for t in range(NSLOT):
                    wait_s(t)
                    pltpu.async_copy(tbl.at[sidx.at[j0 + NSLOT + t]], rows[t], gsems[t])
                return carry

            lax.fori_loop(0, B // NSLOT - 1, grp, 0)
            for t in range(NSLOT):
                wait_g(t)
                pltpu.async_copy(rows[t], acc.at[didx.at[B - NSLOT + t]], ssems[t], add=True)
            for t in range(NSLOT):
                wait_s(t)
        plsc.subcore_barrier()

        for k in range(640 // L):
            pltpu.sync_copy(acc.at[pl.ds(base + k * L, L)], rows0)
            pltpu.sync_copy(rows0, out_hbm.at[pl.ds(base + k * L, L)])

    @pl.when(c == 0)
    def _():
        half(src_go, dst_go, fu0, agg_i0)
        half(src_go, dst_go, fu1, agg_i1)

    @pl.when(c == 1)
    def _():
        half(src_bk, dst_bk, fi0, agg_u0)
        half(src_bk, dst_bk, fi1, agg_u1)


_sc_agg = functools.partial(
    pl.kernel,
    out_type=[jax.ShapeDtypeStruct((ACC_ROWS, H), jnp.float32)] * 4,
    mesh=plsc.VectorSubcoreMesh(core_axis_name="c", subcore_axis_name="s"),
    compiler_params=pltpu.CompilerParams(use_tc_tiling_on_sc=False),
    scratch_types=[
        pltpu.VMEM((B, L), jnp.int32),
        pltpu.VMEM((B, L), jnp.int32),
        *[pltpu.VMEM((L, H), jnp.float32) for _ in range(NSLOT)],
        pltpu.VMEM_SHARED((ACC_ROWS, H), jnp.float32),
        pltpu.VMEM_SHARED((ACC_ROWS, H), jnp.float32),
        *[pltpu.SemaphoreType.DMA for _ in range(2 * NSLOT)],
    ],
)(_sc_body)


def _halves(x):
    xp = jnp.pad(x, ((0, ACC_ROWS - N), (0, 0)))
    return xp[:, :H], xp[:, H:]


def _sc_layer(src_go, dst_go, src_bk, dst_bk, user_feat, item_feat):
    fu0, fu1 = _halves(user_feat)
    fi0, fi1 = _halves(item_feat)
    a_i0, a_i1, a_u0, a_u1 = _sc_agg(src_go, dst_go, src_bk, dst_bk,
                                     fu0, fu1, fi0, fi1)
    agg_i = jnp.concatenate([a_i0[:N], a_i1[:N]], axis=1)
    agg_u = jnp.concatenate([a_u0[:N], a_u1[:N]], axis=1)
    return agg_i, agg_u


def _dense_body(eps_ref, agg_ref, feat_ref, W_ref, b_ref, g_ref, be_ref, out_ref):
    x = (1.0 + eps_ref[0, 0]) * feat_ref[:] + agg_ref[:]
    h = jnp.dot(x, W_ref[:], preferred_element_type=jnp.float32) + b_ref[:]
    m = jnp.mean(h, axis=0, keepdims=True)
    v = jnp.mean((h - m) ** 2, axis=0, keepdims=True)
    xn = (h - m) * lax.rsqrt(v + 1e-5) * g_ref[:] + be_ref[:]
    out_ref[:] = xn * jax.nn.sigmoid(xn)


_dense = pl.pallas_call(
    _dense_body,
    out_shape=jax.ShapeDtypeStruct((N, D), jnp.float32),
)


def _dense_layer(eps, agg, feat, W, b, gamma, beta):
    return _dense(
        jnp.reshape(eps, (1, 1)),
        agg,
        feat,
        W,
        jnp.reshape(b, (1, D)),
        jnp.reshape(gamma, (1, D)),
        jnp.reshape(beta, (1, D)),
    )


def kernel(edge_index_go, edge_index_back, user_emb, item_emb,
           W1_go, b1_go, W1_back, b1_back, W2_go, b2_go, W2_back, b2_back,
           eps1_go, eps1_back, eps2_go, eps2_back,
           gamma1_user, beta1_user, gamma1_item, beta1_item,
           gamma2_user, beta2_user, gamma2_item, beta2_item):
    def prep(col, fill):
        col = col.astype(jnp.int32)
        return jnp.pad(col, (0, EPAD), constant_values=fill).reshape(NTILES, K, L)

    src_go = prep(edge_index_go[0], 0)
    dst_go = prep(edge_index_go[1], N)
    src_bk = prep(edge_index_back[0], 0)
    dst_bk = prep(edge_index_back[1], N)

    agg_i1, agg_u1 = _sc_layer(src_go, dst_go, src_bk, dst_bk, user_emb, item_emb)
    item1 = _dense_layer(eps1_go, agg_i1, item_emb, W1_go, b1_go, gamma1_item, beta1_item)
    user1 = _dense_layer(eps1_back, agg_u1, user_emb, W1_back, b1_back, gamma1_user, beta1_user)

    agg_i2, agg_u2 = _sc_layer(src_go, dst_go, src_bk, dst_bk, user1, item1)
    item2 = _dense_layer(eps2_go, agg_i2, item1, W2_go, b2_go, gamma2_item, beta2_item)
    user2 = _dense_layer(eps2_back, agg_u2, user1, W2_back, b2_back, gamma2_user, beta2_user)

    return jnp.concatenate([user2, item2], axis=0)


# speedup vs baseline: 1.2641x; 1.1704x over previous
import functools

import jax
import jax.numpy as jnp
from jax import lax
from jax.experimental import pallas as pl
from jax.experimental.pallas import tpu as pltpu
from jax.experimental.pallas import tpu_sc as plsc

N = 10000
D = 128
H = 64
E = 320000
NTILES = 16
L = 64
B = 64
NB = 5
NSLOT = 8
K = B * NB
EPAD = NTILES * K * L - E
ACC_ROWS = 10240


def _sc_body(src_go, dst_go, src_bk, dst_bk, feat_u, feat_i,
             agg_i, agg_u, sidx, didx, *scratch):
    rows = scratch[:NSLOT]
    tbl = scratch[NSLOT]
    acc = scratch[NSLOT + 1]
    gsems = scratch[NSLOT + 2:2 * NSLOT + 2]
    ssems = scratch[2 * NSLOT + 2:]
    c = lax.axis_index("c")
    s = lax.axis_index("s")
    rows0 = rows[0]

    zv = jnp.zeros((16,), jnp.float32)

    def zbody(i, carry):
        rows0[i // 4, pl.ds((i % 4) * 16, 16)] = zv
        return carry

    def wait_g(t):
        pltpu.make_async_copy(tbl.at[sidx.at[0]], rows[t], gsems[t]).wait()

    def wait_s(t):
        pltpu.make_async_copy(rows[t], acc.at[didx.at[0]], ssems[t]).wait()

    def half(src_e, dst_e, feat, h, out_hbm):
        base = s * 640
        cs = pl.ds(h * H, H)

        @pl.when(s < NTILES - 1)
        def _():
            pltpu.sync_copy(feat.at[pl.ds(base, 640), cs], tbl.at[pl.ds(base, 640)])

        @pl.when(s == NTILES - 1)
        def _():
            pltpu.sync_copy(feat.at[pl.ds((NTILES - 1) * 640, N - (NTILES - 1) * 640), cs],
                            tbl.at[pl.ds((NTILES - 1) * 640, N - (NTILES - 1) * 640)])

        lax.fori_loop(0, L * 4, zbody, 0)
        for k in range(640 // L):
            pltpu.sync_copy(rows0, acc.at[pl.ds(base + k * L, L)])
        plsc.subcore_barrier()

        for b in range(NB):
            pltpu.sync_copy(src_e.at[s, pl.ds(b * B, B)], sidx)
            pltpu.sync_copy(dst_e.at[s, pl.ds(b * B, B)], didx)
            for t in range(NSLOT):
                pltpu.async_copy(tbl.at[sidx.at[t]], rows[t], gsems[t])

            def grp(q, carry):
                j0 = q * NSLOT
                for t in range(NSLOT):
                    wait_g(t)
                    pltpu.async_copy(rows[t], acc.at[didx.at[j0 + t]], ssems[t], add=True)
                for t in range(NSLOT):
                    wait_s(t)
                    pltpu.async_copy(tbl.at[sidx.at[j0 + NSLOT + t]], rows[t], gsems[t])
                return carry

            lax.fori_loop(0, B // NSLOT - 1, grp, 0)
            for t in range(NSLOT):
                wait_g(t)
                pltpu.async_copy(rows[t], acc.at[didx.at[B - NSLOT + t]], ssems[t], add=True)
            for t in range(NSLOT):
                wait_s(t)
        plsc.subcore_barrier()

        for k in range(640 // L):
            pltpu.sync_copy(acc.at[pl.ds(base + k * L, L)], rows0)
            pltpu.sync_copy(rows0, out_hbm.at[pl.ds(base + k * L, L), cs])

    @pl.when(c == 0)
    def _():
        half(src_go, dst_go, feat_u, 0, agg_i)
        half(src_go, dst_go, feat_u, 1, agg_i)

    @pl.when(c == 1)
    def _():
        half(src_bk, dst_bk, feat_i, 0, agg_u)
        half(src_bk, dst_bk, feat_i, 1, agg_u)


_sc_agg = functools.partial(
    pl.kernel,
    out_type=[jax.ShapeDtypeStruct((ACC_ROWS, D), jnp.float32)] * 2,
    mesh=plsc.VectorSubcoreMesh(core_axis_name="c", subcore_axis_name="s"),
    compiler_params=pltpu.CompilerParams(use_tc_tiling_on_sc=False),
    scratch_types=[
        pltpu.VMEM((B, L), jnp.int32),
        pltpu.VMEM((B, L), jnp.int32),
        *[pltpu.VMEM((L, H), jnp.float32) for _ in range(NSLOT)],
        pltpu.VMEM_SHARED((ACC_ROWS, H), jnp.float32),
        pltpu.VMEM_SHARED((ACC_ROWS, H), jnp.float32),
        *[pltpu.SemaphoreType.DMA for _ in range(2 * NSLOT)],
    ],
)(_sc_body)


def _rel(eps_ref, agg_ref, feat_ref, W_ref, b_ref, g_ref, be_ref):
    x = (1.0 + eps_ref[0, 0]) * feat_ref[:] + agg_ref[pl.ds(0, N), :]
    h = jnp.dot(x, W_ref[:], preferred_element_type=jnp.float32) + b_ref[:]
    m = jnp.mean(h, axis=0, keepdims=True)
    v = jnp.mean((h - m) ** 2, axis=0, keepdims=True)
    xn = (h - m) * lax.rsqrt(v + 1e-5) * g_ref[:] + be_ref[:]
    return xn * jax.nn.sigmoid(xn)


def _dense_mid_body(eps_i, agg_i, feat_i, W_i, b_i, g_i, be_i,
                    eps_u, agg_u, feat_u, W_u, b_u, g_u, be_u,
                    out_i, out_u):
    out_i[:] = _rel(eps_i, agg_i, feat_i, W_i, b_i, g_i, be_i)
    out_u[:] = _rel(eps_u, agg_u, feat_u, W_u, b_u, g_u, be_u)


def _dense_fin_body(eps_i, agg_i, feat_i, W_i, b_i, g_i, be_i,
                    eps_u, agg_u, feat_u, W_u, b_u, g_u, be_u,
                    out):
    out[pl.ds(0, N), :] = _rel(eps_u, agg_u, feat_u, W_u, b_u, g_u, be_u)
    out[pl.ds(N, N), :] = _rel(eps_i, agg_i, feat_i, W_i, b_i, g_i, be_i)


_dense_mid = pl.pallas_call(
    _dense_mid_body,
    out_shape=[jax.ShapeDtypeStruct((N, D), jnp.float32)] * 2,
)

_dense_fin = pl.pallas_call(
    _dense_fin_body,
    out_shape=jax.ShapeDtypeStruct((2 * N, D), jnp.float32),
)


def kernel(edge_index_go, edge_index_back, user_emb, item_emb,
           W1_go, b1_go, W1_back, b1_back, W2_go, b2_go, W2_back, b2_back,
           eps1_go, eps1_back, eps2_go, eps2_back,
           gamma1_user, beta1_user, gamma1_item, beta1_item,
           gamma2_user, beta2_user, gamma2_item, beta2_item):
    def prep(col, fill):
        col = col.astype(jnp.int32)
        return jnp.pad(col, (0, EPAD), constant_values=fill).reshape(NTILES, K, L)

    src_go = prep(edge_index_go[0], 0)
    dst_go = prep(edge_index_go[1], N)
    src_bk = prep(edge_index_back[0], 0)
    dst_bk = prep(edge_index_back[1], N)

    r2 = lambda a: jnp.reshape(a, (1, D))
    s2 = lambda a: jnp.reshape(a, (1, 1))

    agg_i1, agg_u1 = _sc_agg(src_go, dst_go, src_bk, dst_bk, user_emb, item_emb)
    item1, user1 = _dense_mid(
        s2(eps1_go), agg_i1, item_emb, W1_go, r2(b1_go), r2(gamma1_item), r2(beta1_item),
        s2(eps1_back), agg_u1, user_emb, W1_back, r2(b1_back), r2(gamma1_user), r2(beta1_user))

    agg_i2, agg_u2 = _sc_agg(src_go, dst_go, src_bk, dst_bk, user1, item1)
    return _dense_fin(
        s2(eps2_go), agg_i2, item1, W2_go, r2(b2_go), r2(gamma2_item), r2(beta2_item),
        s2(eps2_back), agg_u2, user1, W2_back, r2(b2_back), r2(gamma2_user), r2(beta2_user))


# speedup vs baseline: 1.2796x; 1.0122x over previous
import functools

import jax
import jax.numpy as jnp
from jax import lax
from jax.experimental import pallas as pl
from jax.experimental.pallas import tpu as pltpu
from jax.experimental.pallas import tpu_sc as plsc

N = 10000
D = 128
H = 64
E = 320000
NTILES = 16
L = 64
B = 64
NB = 5
NSLOT = 8
K = B * NB
EPAD = NTILES * K * L - E
ACC_ROWS = 10240


def _sc_body(src_go, dst_go, src_bk, dst_bk, feat_u, feat_i,
             agg_i, agg_u, sidx, didx, *scratch):
    rows = scratch[:NSLOT]
    tbl = scratch[NSLOT]
    acc = scratch[NSLOT + 1]
    gsems = scratch[NSLOT + 2:2 * NSLOT + 2]
    ssems = scratch[2 * NSLOT + 2:]
    c = lax.axis_index("c")
    s = lax.axis_index("s")
    rows0 = rows[0]

    zv = jnp.zeros((16,), jnp.float32)

    def zbody(i, carry):
        rows0[i // 4, pl.ds((i % 4) * 16, 16)] = zv
        return carry

    def wait_g(t):
        pltpu.make_async_copy(tbl.at[sidx.at[0]], rows[t], gsems[t]).wait()

    def wait_s(t):
        pltpu.make_async_copy(rows[t], acc.at[didx.at[0]], ssems[t]).wait()

    def half(src_e, dst_e, feat, h, out_hbm):
        base = s * 640
        cs = pl.ds(h * H, H)

        @pl.when(s < NTILES - 1)
        def _():
            pltpu.sync_copy(feat.at[pl.ds(base, 640), cs], tbl.at[pl.ds(base, 640)])

        @pl.when(s == NTILES - 1)
        def _():
            pltpu.sync_copy(feat.at[pl.ds((NTILES - 1) * 640, N - (NTILES - 1) * 640), cs],
                            tbl.at[pl.ds((NTILES - 1) * 640, N - (NTILES - 1) * 640)])

        lax.fori_loop(0, L * 4, zbody, 0)
        for k in range(640 // L):
            pltpu.async_copy(rows0, acc.at[pl.ds(base + k * L, L)], ssems[k % NSLOT])
        for k in range(640 // L):
            pltpu.make_async_copy(rows0, acc.at[pl.ds(base + k * L, L)],
                                  ssems[k % NSLOT]).wait()
        plsc.subcore_barrier()

        for b in range(NB):
            pltpu.sync_copy(src_e.at[s, pl.ds(b * B, B)], sidx)
            pltpu.sync_copy(dst_e.at[s, pl.ds(b * B, B)], didx)
            for t in range(NSLOT):
                pltpu.async_copy(tbl.at[sidx.at[t]], rows[t], gsems[t])

            def grp(q, carry):
                j0 = q * NSLOT
                for t in range(NSLOT):
                    wait_g(t)
                    pltpu.async_copy(rows[t], acc.at[didx.at[j0 + t]], ssems[t], add=True)
                for t in range(NSLOT):
                    wait_s(t)
                    pltpu.async_copy(tbl.at[sidx.at[j0 + NSLOT + t]], rows[t], gsems[t])
                return carry

            lax.fori_loop(0, B // NSLOT - 1, grp, 0)
            for t in range(NSLOT):
                wait_g(t)
                pltpu.async_copy(rows[t], acc.at[didx.at[B - NSLOT + t]], ssems[t], add=True)
            for t in range(NSLOT):
                wait_s(t)
        plsc.subcore_barrier()

        for k in range(640 // L):
            pltpu.async_copy(acc.at[pl.ds(base + k * L, L)],
                             out_hbm.at[pl.ds(base + k * L, L), cs], gsems[k % NSLOT])
        for k in range(640 // L):
            pltpu.make_async_copy(acc.at[pl.ds(base + k * L, L)],
                                  out_hbm.at[pl.ds(base + k * L, L), cs],
                                  gsems[k % NSLOT]).wait()

    @pl.when(c == 0)
    def _():
        half(src_go, dst_go, feat_u, 0, agg_i)
        half(src_go, dst_go, feat_u, 1, agg_i)

    @pl.when(c == 1)
    def _():
        half(src_bk, dst_bk, feat_i, 0, agg_u)
        half(src_bk, dst_bk, feat_i, 1, agg_u)


_sc_agg = functools.partial(
    pl.kernel,
    out_type=[jax.ShapeDtypeStruct((ACC_ROWS, D), jnp.float32)] * 2,
    mesh=plsc.VectorSubcoreMesh(core_axis_name="c", subcore_axis_name="s"),
    compiler_params=pltpu.CompilerParams(use_tc_tiling_on_sc=False),
    scratch_types=[
        pltpu.VMEM((B, L), jnp.int32),
        pltpu.VMEM((B, L), jnp.int32),
        *[pltpu.VMEM((L, H), jnp.float32) for _ in range(NSLOT)],
        pltpu.VMEM_SHARED((ACC_ROWS, H), jnp.float32),
        pltpu.VMEM_SHARED((ACC_ROWS, H), jnp.float32),
        *[pltpu.SemaphoreType.DMA for _ in range(2 * NSLOT)],
    ],
)(_sc_body)


def _rel(eps_ref, agg_ref, feat_ref, W_ref, b_ref, g_ref, be_ref):
    x = (1.0 + eps_ref[0, 0]) * feat_ref[:] + agg_ref[pl.ds(0, N), :]
    h = jnp.dot(x, W_ref[:], preferred_element_type=jnp.float32) + b_ref[:]
    m = jnp.mean(h, axis=0, keepdims=True)
    v = jnp.mean((h - m) ** 2, axis=0, keepdims=True)
    xn = (h - m) * lax.rsqrt(v + 1e-5) * g_ref[:] + be_ref[:]
    return xn * jax.nn.sigmoid(xn)


def _dense_mid_body(eps_i, agg_i, feat_i, W_i, b_i, g_i, be_i,
                    eps_u, agg_u, feat_u, W_u, b_u, g_u, be_u,
                    out_i, out_u):
    out_i[:] = _rel(eps_i, agg_i, feat_i, W_i, b_i, g_i, be_i)
    out_u[:] = _rel(eps_u, agg_u, feat_u, W_u, b_u, g_u, be_u)


def _dense_fin_body(eps_i, agg_i, feat_i, W_i, b_i, g_i, be_i,
                    eps_u, agg_u, feat_u, W_u, b_u, g_u, be_u,
                    out):
    out[pl.ds(0, N), :] = _rel(eps_u, agg_u, feat_u, W_u, b_u, g_u, be_u)
    out[pl.ds(N, N), :] = _rel(eps_i, agg_i, feat_i, W_i, b_i, g_i, be_i)


_dense_mid = pl.pallas_call(
    _dense_mid_body,
    out_shape=[jax.ShapeDtypeStruct((N, D), jnp.float32)] * 2,
)

_dense_fin = pl.pallas_call(
    _dense_fin_body,
    out_shape=jax.ShapeDtypeStruct((2 * N, D), jnp.float32),
)


def kernel(edge_index_go, edge_index_back, user_emb, item_emb,
           W1_go, b1_go, W1_back, b1_back, W2_go, b2_go, W2_back, b2_back,
           eps1_go, eps1_back, eps2_go, eps2_back,
           gamma1_user, beta1_user, gamma1_item, beta1_item,
           gamma2_user, beta2_user, gamma2_item, beta2_item):
    def prep(col, fill):
        col = col.astype(jnp.int32)
        return jnp.pad(col, (0, EPAD), constant_values=fill).reshape(NTILES, K, L)

    src_go = prep(edge_index_go[0], 0)
    dst_go = prep(edge_index_go[1], N)
    src_bk = prep(edge_index_back[0], 0)
    dst_bk = prep(edge_index_back[1], N)

    r2 = lambda a: jnp.reshape(a, (1, D))
    s2 = lambda a: jnp.reshape(a, (1, 1))

    agg_i1, agg_u1 = _sc_agg(src_go, dst_go, src_bk, dst_bk, user_emb, item_emb)
    item1, user1 = _dense_mid(
        s2(eps1_go), agg_i1, item_emb, W1_go, r2(b1_go), r2(gamma1_item), r2(beta1_item),
        s2(eps1_back), agg_u1, user_emb, W1_back, r2(b1_back), r2(gamma1_user), r2(beta1_user))

    agg_i2, agg_u2 = _sc_agg(src_go, dst_go, src_bk, dst_bk, user1, item1)
    return _dense_fin(
        s2(eps2_go), agg_i2, item1, W2_go, r2(b2_go), r2(gamma2_item), r2(beta2_item),
        s2(eps2_back), agg_u2, user1, W2_back, r2(b2_back), r2(gamma2_user), r2(beta2_user))


# speedup vs baseline: 1.2820x; 1.0019x over previous
import functools

import jax
import jax.numpy as jnp
from jax import lax
from jax.experimental import pallas as pl
from jax.experimental.pallas import tpu as pltpu
from jax.experimental.pallas import tpu_sc as plsc

N = 10000
D = 128
H = 64
E = 320000
NTILES = 16
L = 64
B = 80
NB = 4
NSLOT = 8
K = B * NB
EPAD = NTILES * K * L - E
ACC_ROWS = 10240


def _sc_body(src_go, dst_go, src_bk, dst_bk, feat_u, feat_i,
             agg_i, agg_u, sidx, didx, *scratch):
    rows = scratch[:NSLOT]
    tbl = scratch[NSLOT]
    acc = scratch[NSLOT + 1]
    gsems = scratch[NSLOT + 2:2 * NSLOT + 2]
    ssems = scratch[2 * NSLOT + 2:]
    c = lax.axis_index("c")
    s = lax.axis_index("s")
    rows0 = rows[0]

    zv = jnp.zeros((16,), jnp.float32)

    def zbody(i, carry):
        rows0[i // 4, pl.ds((i % 4) * 16, 16)] = zv
        return carry

    def wait_g(t):
        pltpu.make_async_copy(tbl.at[sidx.at[0]], rows[t], gsems[t]).wait()

    def wait_s(t):
        pltpu.make_async_copy(rows[t], acc.at[didx.at[0]], ssems[t]).wait()

    def half(src_e, dst_e, feat, h, out_hbm):
        base = s * 640
        cs = pl.ds(h * H, H)

        @pl.when(s < NTILES - 1)
        def _():
            pltpu.sync_copy(feat.at[pl.ds(base, 640), cs], tbl.at[pl.ds(base, 640)])

        @pl.when(s == NTILES - 1)
        def _():
            pltpu.sync_copy(feat.at[pl.ds((NTILES - 1) * 640, N - (NTILES - 1) * 640), cs],
                            tbl.at[pl.ds((NTILES - 1) * 640, N - (NTILES - 1) * 640)])

        lax.fori_loop(0, L * 4, zbody, 0)
        for k in range(640 // L):
            pltpu.async_copy(rows0, acc.at[pl.ds(base + k * L, L)], ssems[k % NSLOT])
        for k in range(640 // L):
            pltpu.make_async_copy(rows0, acc.at[pl.ds(base + k * L, L)],
                                  ssems[k % NSLOT]).wait()
        plsc.subcore_barrier()

        for b in range(NB):
            pltpu.sync_copy(src_e.at[s, pl.ds(b * B, B)], sidx)
            pltpu.sync_copy(dst_e.at[s, pl.ds(b * B, B)], didx)
            for t in range(NSLOT):
                pltpu.async_copy(tbl.at[sidx.at[t]], rows[t], gsems[t])

            def grp(q, carry):
                j0 = q * NSLOT
                for t in range(NSLOT):
                    wait_g(t)
                    pltpu.async_copy(rows[t], acc.at[didx.at[j0 + t]], ssems[t], add=True)
                for t in range(NSLOT):
                    wait_s(t)
                    pltpu.async_copy(tbl.at[sidx.at[j0 + NSLOT + t]], rows[t], gsems[t])
                return carry

            lax.fori_loop(0, B // NSLOT - 1, grp, 0)
            for t in range(NSLOT):
                wait_g(t)
                pltpu.async_copy(rows[t], acc.at[didx.at[B - NSLOT + t]], ssems[t], add=True)
            for t in range(NSLOT):
                wait_s(t)
        plsc.subcore_barrier()

        for k in range(640 // L):
            pltpu.async_copy(acc.at[pl.ds(base + k * L, L)],
                             out_hbm.at[pl.ds(base + k * L, L), cs], gsems[k % NSLOT])
        for k in range(640 // L):
            pltpu.make_async_copy(acc.at[pl.ds(base + k * L, L)],
                                  out_hbm.at[pl.ds(base + k * L, L), cs],
                                  gsems[k % NSLOT]).wait()

    @pl.when(c == 0)
    def _():
        half(src_go, dst_go, feat_u, 0, agg_i)
        half(src_go, dst_go, feat_u, 1, agg_i)

    @pl.when(c == 1)
    def _():
        half(src_bk, dst_bk, feat_i, 0, agg_u)
        half(src_bk, dst_bk, feat_i, 1, agg_u)


_sc_agg = functools.partial(
    pl.kernel,
    out_type=[jax.ShapeDtypeStruct((ACC_ROWS, D), jnp.float32)] * 2,
    mesh=plsc.VectorSubcoreMesh(core_axis_name="c", subcore_axis_name="s"),
    compiler_params=pltpu.CompilerParams(use_tc_tiling_on_sc=False),
    scratch_types=[
        pltpu.VMEM((B, L), jnp.int32),
        pltpu.VMEM((B, L), jnp.int32),
        *[pltpu.VMEM((L, H), jnp.float32) for _ in range(NSLOT)],
        pltpu.VMEM_SHARED((ACC_ROWS, H), jnp.float32),
        pltpu.VMEM_SHARED((ACC_ROWS, H), jnp.float32),
        *[pltpu.SemaphoreType.DMA for _ in range(2 * NSLOT)],
    ],
)(_sc_body)


def _rel(eps_ref, agg_ref, feat_ref, W_ref, b_ref, g_ref, be_ref):
    x = (1.0 + eps_ref[0, 0]) * feat_ref[:] + agg_ref[pl.ds(0, N), :]
    h = jnp.dot(x, W_ref[:], preferred_element_type=jnp.float32) + b_ref[:]
    m = jnp.mean(h, axis=0, keepdims=True)
    v = jnp.mean((h - m) ** 2, axis=0, keepdims=True)
    xn = (h - m) * lax.rsqrt(v + 1e-5) * g_ref[:] + be_ref[:]
    return xn * jax.nn.sigmoid(xn)


def _dense_mid_body(eps_i, agg_i, feat_i, W_i, b_i, g_i, be_i,
                    eps_u, agg_u, feat_u, W_u, b_u, g_u, be_u,
                    out_i, out_u):
    out_i[:] = _rel(eps_i, agg_i, feat_i, W_i, b_i, g_i, be_i)
    out_u[:] = _rel(eps_u, agg_u, feat_u, W_u, b_u, g_u, be_u)


def _dense_fin_body(eps_i, agg_i, feat_i, W_i, b_i, g_i, be_i,
                    eps_u, agg_u, feat_u, W_u, b_u, g_u, be_u,
                    out):
    out[pl.ds(0, N), :] = _rel(eps_u, agg_u, feat_u, W_u, b_u, g_u, be_u)
    out[pl.ds(N, N), :] = _rel(eps_i, agg_i, feat_i, W_i, b_i, g_i, be_i)


_dense_mid = pl.pallas_call(
    _dense_mid_body,
    out_shape=[jax.ShapeDtypeStruct((N, D), jnp.float32)] * 2,
)

_dense_fin = pl.pallas_call(
    _dense_fin_body,
    out_shape=jax.ShapeDtypeStruct((2 * N, D), jnp.float32),
)


def kernel(edge_index_go, edge_index_back, user_emb, item_emb,
           W1_go, b1_go, W1_back, b1_back, W2_go, b2_go, W2_back, b2_back,
           eps1_go, eps1_back, eps2_go, eps2_back,
           gamma1_user, beta1_user, gamma1_item, beta1_item,
           gamma2_user, beta2_user, gamma2_item, beta2_item):
    def prep(col, fill):
        col = col.astype(jnp.int32)
        return jnp.pad(col, (0, EPAD), constant_values=fill).reshape(NTILES, K, L)

    src_go = prep(edge_index_go[0], 0)
    dst_go = prep(edge_index_go[1], N)
    src_bk = prep(edge_index_back[0], 0)
    dst_bk = prep(edge_index_back[1], N)

    r2 = lambda a: jnp.reshape(a, (1, D))
    s2 = lambda a: jnp.reshape(a, (1, 1))

    agg_i1, agg_u1 = _sc_agg(src_go, dst_go, src_bk, dst_bk, user_emb, item_emb)
    item1, user1 = _dense_mid(
        s2(eps1_go), agg_i1, item_emb, W1_go, r2(b1_go), r2(gamma1_item), r2(beta1_item),
        s2(eps1_back), agg_u1, user_emb, W1_back, r2(b1_back), r2(gamma1_user), r2(beta1_user))

    agg_i2, agg_u2 = _sc_agg(src_go, dst_go, src_bk, dst_bk, user1, item1)
    return _dense_fin(
        s2(eps2_go), agg_i2, item1, W2_go, r2(b2_go), r2(gamma2_item), r2(beta2_item),
        s2(eps2_back), agg_u2, user1, W2_back, r2(b2_back), r2(gamma2_user), r2(beta2_user))
